# baseline (speedup 1.0000x reference)
import jax
import jax.numpy as jnp
from jax import lax
from jax.experimental import pallas as pl

V_MIN = -10.0
V_MAX = 10.0
NUM_ATOMS = 51
DELTA_Z = (V_MAX - V_MIN) / (NUM_ATOMS - 1)

_ROWS = 16384
_COLS = 64
_BLOCK_R = 512


def _c51_block_kernel(x_ref, out_ref):
    x = x_ref[...]
    t = jnp.clip(x, V_MIN, V_MAX)
    b = (t - V_MIN) / DELTA_Z
    k = lax.broadcasted_iota(
        jnp.int32, (x.shape[0], x.shape[1], NUM_ATOMS), 2
    ).astype(jnp.float32)
    out_ref[...] = jnp.maximum(1.0 - jnp.abs(b[:, :, None] - k), 0.0)


def kernel(scalar):
    return pl.pallas_call(
        _c51_block_kernel,
        grid=(_ROWS // _BLOCK_R,),
        in_specs=[pl.BlockSpec((_BLOCK_R, _COLS), lambda i: (i, 0))],
        out_specs=pl.BlockSpec((_BLOCK_R, _COLS, NUM_ATOMS), lambda i: (i, 0, 0)),
        out_shape=jax.ShapeDtypeStruct((_ROWS, _COLS, NUM_ATOMS), scalar.dtype),
    )(scalar)
